# baseline (speedup 1.0000x reference)
import functools

import jax
import jax.numpy as jnp
from jax.experimental import pallas as pl
from jax.experimental.pallas import tpu as pltpu

N_NEURONS = 1024
N_TIMESTEPS = 20
K = 51
ONE_BITS = 0x3F800000
ROWS_W = 128
CHUNK = 512
STEPS_PER_CHUNK = CHUNK // ROWS_W


def _topk_mask(x):
    xb = jax.lax.bitcast_convert_type(x, jnp.int32)
    r_rows, n = x.shape
    ones = jnp.ones((n, 1), jnp.float32)

    def count(mat_f32, exact=False):
        prec = jax.lax.Precision.HIGHEST if exact else jax.lax.Precision.DEFAULT
        return jnp.dot(mat_f32, ones, precision=prec,
                       preferred_element_type=jnp.float32)

    lo = jnp.zeros((r_rows, 1), jnp.int32)
    w = 1 << 30
    for _ in range(15):
        q = w >> 2
        p1 = lo + q
        p2 = lo + 2 * q
        p3 = lo + 3 * q
        t_a = jnp.where(xb >= p1, 2048.0, 0.0) + jnp.where(xb >= p2, 1.0, 0.0)
        t_b = jnp.where(xb >= p3, 1.0, 0.0)
        a, c3 = count(t_a, exact=True), count(t_b)
        c1 = jnp.floor(a * (1.0 / 2048.0))
        c2 = a - c1 * 2048.0
        lo = jnp.where(c3 >= K, p3,
                       jnp.where(c2 >= K, p2,
                                 jnp.where(c1 >= K, p1, lo)))
        w = q
    thr = lo

    gt = xb > thr
    eq = xb == thr
    c_gt = count(gt.astype(jnp.float32))
    r_need = K - c_gt
    idx = jax.lax.broadcasted_iota(jnp.int32, (r_rows, n), 1)

    lo2 = jnp.zeros((r_rows, 1), jnp.int32)
    w = n
    for _ in range(5):
        q = w >> 2
        c1m = lo2 + (q - 1)
        c2m = lo2 + (2 * q - 1)
        c3m = lo2 + (3 * q - 1)
        t_a = (jnp.where(eq & (idx <= c1m), 2048.0, 0.0)
               + jnp.where(eq & (idx <= c2m), 1.0, 0.0))
        t_b = jnp.where(eq & (idx <= c3m), 1.0, 0.0)
        a, f3 = count(t_a, exact=True), count(t_b)
        f1 = jnp.floor(a * (1.0 / 2048.0))
        f2 = a - f1 * 2048.0
        lo2 = jnp.where(f1 >= r_need, lo2,
                        jnp.where(f2 >= r_need, lo2 + q,
                                  jnp.where(f3 >= r_need, lo2 + 2 * q,
                                            lo2 + 3 * q)))
        w = q
    cutoff = lo2

    return jnp.where(gt | (eq & (idx <= cutoff)), 1.0, 0.0)


def _spike_body(tm_ref, x_ref, o_ref, mask_ref):
    i = pl.program_id(0)

    @pl.when(i % STEPS_PER_CHUNK == 0)
    def _():
        xc = x_ref[...].reshape(CHUNK, x_ref.shape[-1])
        mask_ref[...] = _topk_mask(xc)

    j = i % STEPS_PER_CHUNK
    sl = mask_ref[pl.ds(j * ROWS_W, ROWS_W), :]
    tm = tm_ref[...]
    o_ref[...] = sl[None, None, :, :] * tm[0][None, :, None, None]


@jax.jit
def kernel(features, time_mask):
    batch, seq_len, n = features.shape
    rows = batch * seq_len
    tm = time_mask.astype(features.dtype).reshape(1, N_TIMESTEPS)
    batches_per_chunk = CHUNK // seq_len
    wsteps_per_batch = seq_len // ROWS_W
    grid = (rows // ROWS_W,)
    out = pl.pallas_call(
        _spike_body,
        grid=grid,
        in_specs=[
            pl.BlockSpec((1, N_TIMESTEPS), lambda i: (0, 0)),
            pl.BlockSpec((batches_per_chunk, seq_len, n),
                         lambda i: (i // STEPS_PER_CHUNK, 0, 0)),
        ],
        out_specs=pl.BlockSpec(
            (1, N_TIMESTEPS, ROWS_W, n),
            lambda i: (i // wsteps_per_batch, 0, i % wsteps_per_batch, 0)),
        out_shape=jax.ShapeDtypeStruct(
            (batch, N_TIMESTEPS, seq_len, n), features.dtype),
        scratch_shapes=[pltpu.VMEM((CHUNK, n), jnp.float32)],
    )(tm, features)
    return jnp.transpose(out, (0, 2, 1, 3))


# speedup vs baseline: 3.3137x; 3.3137x over previous
import functools

import jax
import jax.numpy as jnp
from jax.experimental import pallas as pl
from jax.experimental.pallas import tpu as pltpu

N_NEURONS = 1024
N_TIMESTEPS = 20
K = 51
ROWS_W = 128
CHUNK = 512
STEPS_PER_CHUNK = CHUNK // ROWS_W
N_SPLIT = 2


def _topk_mask(x):
    r_rows, n = x.shape
    h = r_rows // N_SPLIT
    xbs = [jax.lax.bitcast_convert_type(x[i * h:(i + 1) * h], jnp.int32)
           for i in range(N_SPLIT)]
    ones = jnp.ones((n, 1), jnp.float32)

    def count(mat_f32):
        return jnp.dot(mat_f32, ones, preferred_element_type=jnp.float32)

    def vbody(_, carry):
        los, his = carry
        mids = [(lo + hi) >> 1 for lo, hi in zip(los, his)]
        cnts = [count((xb >= mid).astype(jnp.float32))
                for xb, mid in zip(xbs, mids)]
        ps = [c >= K for c in cnts]
        new_los = [jnp.where(p, mid, lo) for p, mid, lo in zip(ps, mids, los)]
        new_his = [jnp.where(p, hi, mid) for p, mid, hi in zip(ps, mids, his)]
        return new_los, new_his

    los0 = [jnp.zeros((h, 1), jnp.int32)] * N_SPLIT
    his0 = [jnp.full((h, 1), 1 << 30, jnp.int32)] * N_SPLIT
    thrs, _ = jax.lax.fori_loop(0, 30, vbody, (los0, his0))

    gts = [xb > thr for xb, thr in zip(xbs, thrs)]
    eqs = [xb == thr for xb, thr in zip(xbs, thrs)]
    r_needs = [K - count(gt.astype(jnp.float32)) for gt in gts]
    idx = jax.lax.broadcasted_iota(jnp.int32, (h, n), 1)

    def ibody(_, carry):
        los2, his2 = carry
        mids = [(lo + hi) >> 1 for lo, hi in zip(los2, his2)]
        cnts = [count(jnp.where(eq & (idx <= mid), 1.0, 0.0))
                for eq, mid in zip(eqs, mids)]
        ps = [c >= r for c, r in zip(cnts, r_needs)]
        new_los = [jnp.where(p, lo, mid) for p, mid, lo in zip(ps, mids, los2)]
        new_his = [jnp.where(p, mid, hi) for p, mid, hi in zip(ps, mids, his2)]
        return new_los, new_his

    los2_0 = [jnp.full((h, 1), -1, jnp.int32)] * N_SPLIT
    his2_0 = [jnp.full((h, 1), n - 1, jnp.int32)] * N_SPLIT
    _, cutoffs = jax.lax.fori_loop(0, 10, ibody, (los2_0, his2_0))

    parts = [jnp.where(gt | (eq & (idx <= cut)), 1.0, 0.0)
             for gt, eq, cut in zip(gts, eqs, cutoffs)]
    return jnp.concatenate(parts, axis=0)


def _spike_body(tm_ref, x_ref, o_ref, mask_ref):
    i = pl.program_id(0)

    @pl.when(i % STEPS_PER_CHUNK == 0)
    def _():
        xc = x_ref[...].reshape(CHUNK, x_ref.shape[-1])
        mask_ref[...] = _topk_mask(xc)

    j = i % STEPS_PER_CHUNK
    sl = mask_ref[pl.ds(j * ROWS_W, ROWS_W), :]
    tm = tm_ref[...]
    o_ref[...] = sl[None, None, :, :] * tm[0][None, :, None, None]


@jax.jit
def kernel(features, time_mask):
    batch, seq_len, n = features.shape
    rows = batch * seq_len
    tm = time_mask.astype(features.dtype).reshape(1, N_TIMESTEPS)
    batches_per_chunk = CHUNK // seq_len
    wsteps_per_batch = seq_len // ROWS_W
    grid = (rows // ROWS_W,)
    out = pl.pallas_call(
        _spike_body,
        grid=grid,
        in_specs=[
            pl.BlockSpec((1, N_TIMESTEPS), lambda i: (0, 0)),
            pl.BlockSpec((batches_per_chunk, seq_len, n),
                         lambda i: (i // STEPS_PER_CHUNK, 0, 0)),
        ],
        out_specs=pl.BlockSpec(
            (1, N_TIMESTEPS, ROWS_W, n),
            lambda i: (i // wsteps_per_batch, 0, i % wsteps_per_batch, 0)),
        out_shape=jax.ShapeDtypeStruct(
            (batch, N_TIMESTEPS, seq_len, n), features.dtype),
        scratch_shapes=[pltpu.VMEM((CHUNK, n), jnp.float32)],
    )(tm, features)
    return jnp.transpose(out, (0, 2, 1, 3))


# speedup vs baseline: 3.9447x; 1.1904x over previous
import functools

import jax
import jax.numpy as jnp
from jax.experimental import pallas as pl
from jax.experimental.pallas import tpu as pltpu

N_NEURONS = 1024
N_TIMESTEPS = 20
K = 51
ROWS_W = 128
CHUNK = 512
STEPS_PER_CHUNK = CHUNK // ROWS_W
N_SPLIT = 2


def _topk_mask(x):
    r_rows, n = x.shape
    h = r_rows // N_SPLIT
    xbs = [jax.lax.bitcast_convert_type(x[i * h:(i + 1) * h], jnp.int32)
           for i in range(N_SPLIT)]
    ones = jnp.ones((n, 1), jnp.float32)

    def count(mat_f32):
        return jnp.dot(mat_f32, ones, preferred_element_type=jnp.float32)

    los = [jnp.zeros((h, 1), jnp.int32)] * N_SPLIT
    w = 1 << 30
    for _ in range(15):
        q = w >> 2
        new_los = []
        for xb, lo in zip(xbs, los):
            p1 = lo + q
            p2 = lo + 2 * q
            p3 = lo + 3 * q
            c1 = count(jnp.where(xb >= p1, 1.0, 0.0))
            c2 = count(jnp.where(xb >= p2, 1.0, 0.0))
            c3 = count(jnp.where(xb >= p3, 1.0, 0.0))
            new_los.append(jnp.where(c3 >= K, p3,
                           jnp.where(c2 >= K, p2,
                                     jnp.where(c1 >= K, p1, lo))))
        los = new_los
        w = q
    thrs = los

    gts = [xb > thr for xb, thr in zip(xbs, thrs)]
    eqs = [xb == thr for xb, thr in zip(xbs, thrs)]
    r_needs = [K - count(gt.astype(jnp.float32)) for gt in gts]
    idx = jax.lax.broadcasted_iota(jnp.int32, (h, n), 1)

    los2 = [jnp.zeros((h, 1), jnp.int32)] * N_SPLIT
    w = n
    for _ in range(5):
        q = w >> 2
        new_los2 = []
        for eq, r_need, lo2 in zip(eqs, r_needs, los2):
            c1m = lo2 + (q - 1)
            c2m = lo2 + (2 * q - 1)
            c3m = lo2 + (3 * q - 1)
            f1 = count(jnp.where(eq & (idx <= c1m), 1.0, 0.0))
            f2 = count(jnp.where(eq & (idx <= c2m), 1.0, 0.0))
            f3 = count(jnp.where(eq & (idx <= c3m), 1.0, 0.0))
            new_los2.append(
                jnp.where(f1 >= r_need, lo2,
                          jnp.where(f2 >= r_need, lo2 + q,
                                    jnp.where(f3 >= r_need, lo2 + 2 * q,
                                              lo2 + 3 * q))))
        los2 = new_los2
        w = q
    cutoffs = los2

    parts = [jnp.where(gt | (eq & (idx <= cut)), 1.0, 0.0)
             for gt, eq, cut in zip(gts, eqs, cutoffs)]
    return jnp.concatenate(parts, axis=0)


def _spike_body(tm_ref, x_ref, o_ref, mask_ref):
    i = pl.program_id(0)

    @pl.when(i % STEPS_PER_CHUNK == 0)
    def _():
        xc = x_ref[...].reshape(CHUNK, x_ref.shape[-1])
        mask_ref[...] = _topk_mask(xc)

    j = i % STEPS_PER_CHUNK
    sl = mask_ref[pl.ds(j * ROWS_W, ROWS_W), :]
    tm = tm_ref[...]
    o_ref[...] = sl[None, None, :, :] * tm[0][None, :, None, None]


@jax.jit
def kernel(features, time_mask):
    batch, seq_len, n = features.shape
    rows = batch * seq_len
    tm = time_mask.astype(features.dtype).reshape(1, N_TIMESTEPS)
    batches_per_chunk = CHUNK // seq_len
    wsteps_per_batch = seq_len // ROWS_W
    grid = (rows // ROWS_W,)
    out = pl.pallas_call(
        _spike_body,
        grid=grid,
        in_specs=[
            pl.BlockSpec((1, N_TIMESTEPS), lambda i: (0, 0)),
            pl.BlockSpec((batches_per_chunk, seq_len, n),
                         lambda i: (i // STEPS_PER_CHUNK, 0, 0)),
        ],
        out_specs=pl.BlockSpec(
            (1, N_TIMESTEPS, ROWS_W, n),
            lambda i: (i // wsteps_per_batch, 0, i % wsteps_per_batch, 0)),
        out_shape=jax.ShapeDtypeStruct(
            (batch, N_TIMESTEPS, seq_len, n), features.dtype),
        scratch_shapes=[pltpu.VMEM((CHUNK, n), jnp.float32)],
    )(tm, features)
    return jnp.transpose(out, (0, 2, 1, 3))


# speedup vs baseline: 4.4837x; 1.1367x over previous
import functools

import jax
import jax.numpy as jnp
from jax.experimental import pallas as pl
from jax.experimental.pallas import tpu as pltpu

N_NEURONS = 1024
N_TIMESTEPS = 20
K = 51
ROWS_W = 128
CHUNK = 512
STEPS_PER_CHUNK = CHUNK // ROWS_W
N_SPLIT = 2


def _topk_mask(x):
    r_rows, n = x.shape
    h = r_rows // N_SPLIT
    xbs = [jax.lax.bitcast_convert_type(x[i * h:(i + 1) * h], jnp.int32)
           for i in range(N_SPLIT)]
    ones = jnp.ones((n, 1), jnp.float32)

    def count(mat_f32):
        return jnp.dot(mat_f32, ones, preferred_element_type=jnp.float32)

    los = [jnp.zeros((h, 1), jnp.int32)] * N_SPLIT
    w = 1 << 30
    for _ in range(15):
        q = w >> 2
        new_los = []
        for xb, lo in zip(xbs, los):
            p1 = lo + q
            p2 = lo + 2 * q
            p3 = lo + 3 * q
            c1 = count(jnp.where(xb >= p1, 1.0, 0.0))
            c2 = count(jnp.where(xb >= p2, 1.0, 0.0))
            c3 = count(jnp.where(xb >= p3, 1.0, 0.0))
            new_los.append(jnp.where(c3 >= K, p3,
                           jnp.where(c2 >= K, p2,
                                     jnp.where(c1 >= K, p1, lo))))
        los = new_los
        w = q
    thrs = los

    gts = [xb > thr for xb, thr in zip(xbs, thrs)]
    eqs = [xb == thr for xb, thr in zip(xbs, thrs)]
    r_needs = [K - count(gt.astype(jnp.float32)) for gt in gts]
    idx = jax.lax.broadcasted_iota(jnp.int32, (h, n), 1)

    c_eqs = [count(eq.astype(jnp.float32)) for eq in eqs]
    excess = jnp.maximum(*[jnp.max(c_eq - r_need)
                           for c_eq, r_need in zip(c_eqs, r_needs)])

    def _tie_search():
        los2 = [jnp.zeros((h, 1), jnp.int32)] * N_SPLIT
        w = n
        for _ in range(5):
            q = w >> 2
            new_los2 = []
            for eq, r_need, lo2 in zip(eqs, r_needs, los2):
                c1m = lo2 + (q - 1)
                c2m = lo2 + (2 * q - 1)
                c3m = lo2 + (3 * q - 1)
                f1 = count(jnp.where(eq & (idx <= c1m), 1.0, 0.0))
                f2 = count(jnp.where(eq & (idx <= c2m), 1.0, 0.0))
                f3 = count(jnp.where(eq & (idx <= c3m), 1.0, 0.0))
                new_los2.append(
                    jnp.where(f1 >= r_need, lo2,
                              jnp.where(f2 >= r_need, lo2 + q,
                                        jnp.where(f3 >= r_need, lo2 + 2 * q,
                                                  lo2 + 3 * q))))
            los2 = new_los2
            w = q
        return los2

    def _no_ties():
        return [jnp.full((h, 1), n - 1, jnp.int32)] * N_SPLIT

    cutoffs = jax.lax.cond(excess > 0.0, _tie_search, _no_ties)

    parts = [jnp.where(gt | (eq & (idx <= cut)), 1.0, 0.0)
             for gt, eq, cut in zip(gts, eqs, cutoffs)]
    return jnp.concatenate(parts, axis=0)


def _spike_body(tm_ref, x_ref, o_ref, mask_ref):
    i = pl.program_id(0)

    @pl.when(i % STEPS_PER_CHUNK == 0)
    def _():
        xc = x_ref[...].reshape(CHUNK, x_ref.shape[-1])
        mask_ref[...] = _topk_mask(xc)

    j = i % STEPS_PER_CHUNK
    sl = mask_ref[pl.ds(j * ROWS_W, ROWS_W), :]
    tm = tm_ref[...]
    o_ref[...] = sl[None, None, :, :] * tm[0][None, :, None, None]


@jax.jit
def kernel(features, time_mask):
    batch, seq_len, n = features.shape
    rows = batch * seq_len
    tm = time_mask.astype(features.dtype).reshape(1, N_TIMESTEPS)
    batches_per_chunk = CHUNK // seq_len
    wsteps_per_batch = seq_len // ROWS_W
    grid = (rows // ROWS_W,)
    out = pl.pallas_call(
        _spike_body,
        grid=grid,
        in_specs=[
            pl.BlockSpec((1, N_TIMESTEPS), lambda i: (0, 0)),
            pl.BlockSpec((batches_per_chunk, seq_len, n),
                         lambda i: (i // STEPS_PER_CHUNK, 0, 0)),
        ],
        out_specs=pl.BlockSpec(
            (1, N_TIMESTEPS, ROWS_W, n),
            lambda i: (i // wsteps_per_batch, 0, i % wsteps_per_batch, 0)),
        out_shape=jax.ShapeDtypeStruct(
            (batch, N_TIMESTEPS, seq_len, n), features.dtype),
        scratch_shapes=[pltpu.VMEM((CHUNK, n), jnp.float32)],
    )(tm, features)
    return jnp.transpose(out, (0, 2, 1, 3))


# speedup vs baseline: 4.6149x; 1.0293x over previous
import functools

import jax
import jax.numpy as jnp
from jax.experimental import pallas as pl
from jax.experimental.pallas import tpu as pltpu

N_NEURONS = 1024
N_TIMESTEPS = 20
K = 51
ROWS_W = 128
CHUNK = 256
STEPS_PER_CHUNK = CHUNK // ROWS_W
N_SPLIT = 2


def _topk_mask(x):
    r_rows, n = x.shape
    h = r_rows // N_SPLIT
    xbs = [jax.lax.bitcast_convert_type(x[i * h:(i + 1) * h], jnp.int32)
           for i in range(N_SPLIT)]
    ones = jnp.ones((n, 1), jnp.float32)

    def count(mat_f32):
        return jnp.dot(mat_f32, ones, preferred_element_type=jnp.float32)

    los = [jnp.zeros((h, 1), jnp.int32)] * N_SPLIT
    w = 1 << 30
    for _ in range(15):
        q = w >> 2
        new_los = []
        for xb, lo in zip(xbs, los):
            p1 = lo + q
            p2 = lo + 2 * q
            p3 = lo + 3 * q
            c1 = count(jnp.where(xb >= p1, 1.0, 0.0))
            c2 = count(jnp.where(xb >= p2, 1.0, 0.0))
            c3 = count(jnp.where(xb >= p3, 1.0, 0.0))
            new_los.append(jnp.where(c3 >= K, p3,
                           jnp.where(c2 >= K, p2,
                                     jnp.where(c1 >= K, p1, lo))))
        los = new_los
        w = q
    thrs = los

    gts = [xb > thr for xb, thr in zip(xbs, thrs)]
    eqs = [xb == thr for xb, thr in zip(xbs, thrs)]
    r_needs = [K - count(gt.astype(jnp.float32)) for gt in gts]
    idx = jax.lax.broadcasted_iota(jnp.int32, (h, n), 1)

    c_eqs = [count(eq.astype(jnp.float32)) for eq in eqs]
    excess = jnp.maximum(*[jnp.max(c_eq - r_need)
                           for c_eq, r_need in zip(c_eqs, r_needs)])

    def _tie_search():
        los2 = [jnp.zeros((h, 1), jnp.int32)] * N_SPLIT
        w = n
        for _ in range(5):
            q = w >> 2
            new_los2 = []
            for eq, r_need, lo2 in zip(eqs, r_needs, los2):
                c1m = lo2 + (q - 1)
                c2m = lo2 + (2 * q - 1)
                c3m = lo2 + (3 * q - 1)
                f1 = count(jnp.where(eq & (idx <= c1m), 1.0, 0.0))
                f2 = count(jnp.where(eq & (idx <= c2m), 1.0, 0.0))
                f3 = count(jnp.where(eq & (idx <= c3m), 1.0, 0.0))
                new_los2.append(
                    jnp.where(f1 >= r_need, lo2,
                              jnp.where(f2 >= r_need, lo2 + q,
                                        jnp.where(f3 >= r_need, lo2 + 2 * q,
                                                  lo2 + 3 * q))))
            los2 = new_los2
            w = q
        return los2

    def _no_ties():
        return [jnp.full((h, 1), n - 1, jnp.int32)] * N_SPLIT

    cutoffs = jax.lax.cond(excess > 0.0, _tie_search, _no_ties)

    parts = [jnp.where(gt | (eq & (idx <= cut)), 1.0, 0.0)
             for gt, eq, cut in zip(gts, eqs, cutoffs)]
    return jnp.concatenate(parts, axis=0)


def _spike_body(tm_ref, x_ref, o_ref, mask_ref):
    i = pl.program_id(0)

    @pl.when(i % STEPS_PER_CHUNK == 0)
    def _():
        xc = x_ref[...].reshape(CHUNK, x_ref.shape[-1])
        mask_ref[...] = _topk_mask(xc)

    j = i % STEPS_PER_CHUNK
    sl = mask_ref[pl.ds(j * ROWS_W, ROWS_W), :]
    tm = tm_ref[...]
    o_ref[...] = sl[None, None, :, :] * tm[0][None, :, None, None]


@jax.jit
def kernel(features, time_mask):
    batch, seq_len, n = features.shape
    rows = batch * seq_len
    tm = time_mask.astype(features.dtype).reshape(1, N_TIMESTEPS)
    batches_per_chunk = CHUNK // seq_len
    wsteps_per_batch = seq_len // ROWS_W
    grid = (rows // ROWS_W,)
    out = pl.pallas_call(
        _spike_body,
        grid=grid,
        in_specs=[
            pl.BlockSpec((1, N_TIMESTEPS), lambda i: (0, 0)),
            pl.BlockSpec((batches_per_chunk, seq_len, n),
                         lambda i: (i // STEPS_PER_CHUNK, 0, 0)),
        ],
        out_specs=pl.BlockSpec(
            (1, N_TIMESTEPS, ROWS_W, n),
            lambda i: (i // wsteps_per_batch, 0, i % wsteps_per_batch, 0)),
        out_shape=jax.ShapeDtypeStruct(
            (batch, N_TIMESTEPS, seq_len, n), features.dtype),
        scratch_shapes=[pltpu.VMEM((CHUNK, n), jnp.float32)],
    )(tm, features)
    return jnp.transpose(out, (0, 2, 1, 3))
